# baseline (speedup 1.0000x reference)
import jax
import jax.numpy as jnp
from jax.experimental import pallas as pl
import jax.experimental.pallas.tpu as pltpu

N_UAV = 1024
N_TGT = 512
F_UAV = 128
F_TGT = 64
HID = 64
HEADS = 4
_BN_EPS = 1e-5
_NEG_SLOPE = 0.2


def _fused_kernel(uf_ref, tf_ref, adj_hbm, tadj_hbm, W1_ref, as1_ref, ad1_ref,
                  b1_ref, W2_ref, as2_ref, ad2_ref, b2_ref, bn1g_ref, bn1b_ref,
                  bn2g_ref, bn2b_ref, Wt_ref, bt_ref, tbng_ref, tbnb_ref,
                  Wf1_ref, bf1_ref, Wf2_ref, bf2_ref, out_ref,
                  adj_vmem, tadj_vmem, adj_sem, tadj_sem):
    f32 = jnp.float32
    N = N_UAV

    adj_cp = pltpu.make_async_copy(adj_hbm, adj_vmem, adj_sem)
    tadj_cp = pltpu.make_async_copy(tadj_hbm, tadj_vmem, tadj_sem)
    adj_cp.start()
    tadj_cp.start()

    def bn(x, g, b):
        m = jnp.mean(x, axis=0, keepdims=True)
        v = jnp.mean((x - m) ** 2, axis=0, keepdims=True)
        return (x - m) / jnp.sqrt(v + _BN_EPS) * g + b

    def elu(x):
        return jnp.where(x > 0, x, jnp.exp(x) - 1.0)

    t0 = jnp.dot(tf_ref[...], Wt_ref[...], preferred_element_type=f32)
    th = jnp.maximum(bn(t0 + bt_ref[...], tbng_ref[...], tbnb_ref[...]), 0.0)

    ones_src = jnp.ones((N, 1), f32)

    def gat_pre(x, W, a_src, a_dst, heads, hid):
        h = jnp.dot(x, W, preferred_element_type=f32)
        pre = []
        for k in range(heads):
            hcol = h[:, k * hid:(k + 1) * hid]
            al = jax.lax.dot_general(hcol, a_src[k:k + 1, :],
                                     (((1,), (1,)), ((), ())),
                                     preferred_element_type=f32)
            ar = jax.lax.dot_general(a_dst[k:k + 1, :], hcol,
                                     (((1,), (1,)), ((), ())),
                                     preferred_element_type=f32)
            pre.append((hcol, al, ar))
        return pre

    def gat_post(pre, neg_mask, heads, hid):
        cols = []
        for hcol, al, ar in pre:
            e = (al + ar) + neg_mask
            e = jnp.maximum(e, _NEG_SLOPE * e)
            b = jnp.max(al, axis=0, keepdims=True) + ar
            emax = jnp.maximum(b, _NEG_SLOPE * b)
            ex = jnp.exp(e - emax)
            nd = jax.lax.dot_general(ex, jnp.concatenate([hcol, ones_src], 1),
                                     (((0,), (0,)), ((), ())),
                                     preferred_element_type=f32)
            inv = 1.0 / (nd[:, hid:hid + 1] + 1e-16)
            cols.append(nd[:, :hid] * inv)
        return jnp.concatenate(cols, axis=1) if len(cols) > 1 else cols[0]

    pre1 = gat_pre(uf_ref[...], W1_ref[...], as1_ref[...], ad1_ref[...],
                   HEADS, HID)

    adj_cp.wait()
    adj = adj_vmem[...]
    drow = jax.lax.broadcasted_iota(jnp.int32, (N, N), 0)
    dcol = jax.lax.broadcasted_iota(jnp.int32, (N, N), 1)
    diag = drow == dcol
    valid = jnp.logical_or(jnp.logical_and(adj != 0.0, jnp.logical_not(diag)),
                           diag)
    neg_mask = jnp.where(valid, 0.0, -jnp.inf)

    x1 = gat_post(pre1, neg_mask, HEADS, HID)
    x1 = elu(bn(x1 + b1_ref[...], bn1g_ref[...], bn1b_ref[...]))

    pre2 = gat_pre(x1, W2_ref[...], as2_ref[...], ad2_ref[...], 1, HID)
    x2 = gat_post(pre2, neg_mask, 1, HID)
    uav_h = elu(bn(x2 + b2_ref[...], bn2g_ref[...], bn2b_ref[...]))

    tadj_cp.wait()
    vis = (tadj_vmem[...] > 0).astype(f32)
    cnt = jax.lax.dot_general(vis, jnp.ones((N_TGT, 1), f32),
                              (((1,), (0,)), ((), ())),
                              preferred_element_type=f32)
    pooled = jnp.dot(vis, th, preferred_element_type=f32)
    tfeat = jnp.where(cnt > 0, pooled / jnp.maximum(cnt, 1.0), 0.0)

    comb = jnp.concatenate([uav_h, tfeat], axis=1)
    hidden = jnp.maximum(
        jnp.dot(comb, Wf1_ref[...], preferred_element_type=f32) + bf1_ref[...],
        0.0)
    out_ref[...] = (jnp.dot(hidden, Wf2_ref[...], preferred_element_type=f32)
                    + bf2_ref[...])


@jax.jit
def kernel(uav_features, target_features, uav_adj, target_adj, W1, att_src1,
           att_dst1, b1, W2, att_src2, att_dst2, b2, bn1_g, bn1_b, bn2_g,
           bn2_b, Wt, bt, tbn_g, tbn_b, Wf1, bf1, Wf2, bf2):
    row = lambda a: a.reshape(1, -1)
    vmem = pl.BlockSpec(memory_space=pltpu.MemorySpace.VMEM)
    hbm = pl.BlockSpec(memory_space=pltpu.MemorySpace.HBM)
    specs = [vmem, vmem, hbm, hbm] + [vmem] * 20
    return pl.pallas_call(
        _fused_kernel,
        out_shape=jax.ShapeDtypeStruct((N_UAV, HID // 2), jnp.float32),
        in_specs=specs,
        scratch_shapes=[
            pltpu.VMEM((N_UAV, N_UAV), jnp.float32),
            pltpu.VMEM((N_UAV, N_TGT), jnp.float32),
            pltpu.SemaphoreType.DMA,
            pltpu.SemaphoreType.DMA,
        ],
        compiler_params=pltpu.CompilerParams(
            vmem_limit_bytes=100 * 1024 * 1024),
    )(uav_features, target_features, uav_adj, target_adj, W1, att_src1,
      att_dst1, row(b1), W2, att_src2, att_dst2, row(b2), row(bn1_g),
      row(bn1_b), row(bn2_g), row(bn2_b), Wt, row(bt), row(tbn_g), row(tbn_b),
      Wf1, row(bf1), Wf2, row(bf2))
